# baseline (speedup 1.0000x reference)
import jax
import jax.numpy as jnp
from jax import lax
from jax.experimental import pallas as pl
from jax.experimental.pallas import tpu as pltpu
from jax.experimental.pallas import tpu_sc as plsc

_NC = 2
_NS = 16
_L = 16
_NW = _NC * _NS
_B = 16384
_E = 1000000
_D = 32
_BPW = _B // _NW
_CH = 128
_NCH = _BPW // _CH
_G = _CH // _L
_NT = _E // _CH
_TAIL = _E - _NT * _CH


def _trig_body(r_ref, c_ref, s_ref):
    c_ref[...] = jnp.cos(r_ref[...])
    s_ref[...] = jnp.sin(r_ref[...])


def _trig_tables(rel):
    cos_t, sin_t = pl.pallas_call(
        _trig_body,
        out_shape=(
            jax.ShapeDtypeStruct(rel.shape, rel.dtype),
            jax.ShapeDtypeStruct(rel.shape, rel.dtype),
        ),
    )(rel)
    return jnp.concatenate([cos_t, sin_t], axis=1).reshape(500, 128)


_W = _CH * 2 * _D


def _transpose_tile(buf, tbuf):
    lane64 = lax.iota(jnp.int32, _L) * (2 * _D)

    for j in range(2 * _D):
        for c0 in range(0, _CH, _L):
            vals = buf[j, pl.ds(c0, _L)]
            plsc.store_scatter(tbuf, [lane64 + (j + c0 * 2 * _D)], vals)


def _fire_reads(entT_hbm, buf, t, sem):
    for p in range(8):
        pltpu.async_copy(
            entT_hbm.at[pl.ds(p * 8, 8), pl.ds(t * _CH, _CH)],
            buf.at[pl.ds(p * 8, 8)], sem)


def _drain(src, dst, sem):
    pltpu.make_async_copy(src, dst, sem).wait()


def _detile_body(entT_hbm, tail_hbm, out_hbm,
                 buf_a, buf_b, tbuf_a, tbuf_b, tailbuf,
                 rsem_a, rsem_b, wsem_a, wsem_b):
    wid = lax.axis_index("s") * _NC + lax.axis_index("c")
    nt = jnp.int32(_NT // _NW) + (wid < _NT % _NW).astype(jnp.int32)
    dummy = entT_hbm.at[pl.ds(0, 64), pl.ds(0, _CH)]

    _fire_reads(entT_hbm, buf_a, wid, rsem_a)

    def pipe(i, carry):
        q0 = i * 2
        q1 = q0 + 1
        q2 = q0 + 2

        @pl.when(q1 < nt)
        def _():
            _fire_reads(entT_hbm, buf_b, q1 * _NW + wid, rsem_b)

        _drain(dummy, buf_a, rsem_a)

        @pl.when(i > 0)
        def _():
            _drain(tbuf_a, out_hbm.at[pl.ds(0, _W)], wsem_a)

        _transpose_tile(buf_a, tbuf_a)
        pltpu.async_copy(tbuf_a, out_hbm.at[pl.ds((q0 * _NW + wid) * _W, _W)],
                         wsem_a)

        @pl.when(q2 < nt)
        def _():
            _fire_reads(entT_hbm, buf_a, q2 * _NW + wid, rsem_a)

        @pl.when(q1 < nt)
        def _():
            _drain(dummy, buf_b, rsem_b)

            @pl.when(i > 0)
            def _():
                _drain(tbuf_b, out_hbm.at[pl.ds(0, _W)], wsem_b)

            _transpose_tile(buf_b, tbuf_b)
            pltpu.async_copy(
                tbuf_b, out_hbm.at[pl.ds((q1 * _NW + wid) * _W, _W)], wsem_b)

        return carry

    lax.fori_loop(0, (nt + 1) // 2, pipe, 0)
    _drain(tbuf_a, out_hbm.at[pl.ds(0, _W)], wsem_a)
    _drain(tbuf_b, out_hbm.at[pl.ds(0, _W)], wsem_b)

    @pl.when(wid == _NW - 1)
    def _tail():
        pltpu.sync_copy(tail_hbm, tailbuf)
        pltpu.sync_copy(tailbuf, out_hbm.at[pl.ds(_NT * _W, _TAIL * 2 * _D)])


def _detile(entT, tail_flat):
    mesh = plsc.VectorSubcoreMesh(
        core_axis_name="c", subcore_axis_name="s",
        num_cores=_NC, num_subcores=_NS,
    )
    return pl.kernel(
        _detile_body,
        out_type=jax.ShapeDtypeStruct((_E * 2 * _D,), jnp.float32),
        mesh=mesh,
        compiler_params=pltpu.CompilerParams(
            needs_layout_passes=False, use_tc_tiling_on_sc=True),
        scratch_types=[
            pltpu.VMEM((2 * _D, _CH), jnp.float32),
            pltpu.VMEM((2 * _D, _CH), jnp.float32),
            pltpu.VMEM((_W,), jnp.float32),
            pltpu.VMEM((_W,), jnp.float32),
            pltpu.VMEM((_TAIL * 2 * _D,), jnp.float32),
            pltpu.SemaphoreType.DMA,
            pltpu.SemaphoreType.DMA,
            pltpu.SemaphoreType.DMA,
            pltpu.SemaphoreType.DMA,
        ],
    )(entT, tail_flat)


def _score_body(hidx_hbm, ridx_hbm, tidx_hbm, ent_hbm, cs_hbm, out_hbm,
                hidx_v, tidx_v, ridx_v, rsh_v,
                hrows, trows, csrows, out_v, sem, rsem):
    wid = lax.axis_index("s") * _NC + lax.axis_index("c")
    row0 = wid * _NCH
    pltpu.sync_copy(hidx_hbm.at[pl.ds(row0, _NCH)], hidx_v)
    pltpu.sync_copy(tidx_hbm.at[pl.ds(row0, _NCH)], tidx_v)
    pltpu.sync_copy(ridx_hbm.at[pl.ds(row0, _NCH)], ridx_v)
    for k in range(_NCH):
        for g in range(_G):
            sl = pl.ds(g * _L, _L)
            rsh_v[k, sl] = lax.shift_right_logical(ridx_v[k, sl], 1)

    lane = lax.iota(jnp.int32, _L)
    for k in range(_NCH):
        copies = [
            pltpu.async_copy(ent_hbm.at[hidx_v.at[k]], hrows, sem),
            pltpu.async_copy(ent_hbm.at[tidx_v.at[k]], trows, sem),
            pltpu.async_copy(cs_hbm.at[rsh_v.at[k]], csrows, rsem),
        ]
        for cp in copies:
            cp.wait()

        for g in range(_G):
            sl = pl.ds(g * _L, _L)
            rows = g * _L + lane
            pr = (ridx_v[k, sl] & 1) * 64

            def dbody(d, acc, rows=rows, pr=pr):
                dd = jnp.full((_L,), 0, jnp.int32) + d
                re = plsc.load_gather(hrows, [rows, dd])
                im = plsc.load_gather(hrows, [rows, dd + _D])
                tre = plsc.load_gather(trows, [rows, dd])
                tim = plsc.load_gather(trows, [rows, dd + _D])
                c = plsc.load_gather(csrows, [rows, pr + d])
                s = plsc.load_gather(csrows, [rows, pr + d + _D])
                return (acc + jnp.abs(re * c - im * s - tre)
                        + jnp.abs(re * s + im * c - tim))

            acc = lax.fori_loop(0, _D, dbody, jnp.zeros((_L,), jnp.float32))
            out_v[pl.ds(k * _CH + g * _L, _L)] = -acc
    pltpu.sync_copy(out_v, out_hbm.at[pl.ds(wid * _BPW, _BPW)])


def _score(hidx2d, ridx2d, tidx2d, ent_lin, cs):
    mesh = plsc.VectorSubcoreMesh(
        core_axis_name="c", subcore_axis_name="s",
        num_cores=_NC, num_subcores=_NS,
    )
    return pl.kernel(
        _score_body,
        out_type=jax.ShapeDtypeStruct((_B,), jnp.float32),
        mesh=mesh,
        compiler_params=pltpu.CompilerParams(
            needs_layout_passes=False, use_tc_tiling_on_sc=False),
        scratch_types=[
            pltpu.VMEM((_NCH, _CH), jnp.int32),
            pltpu.VMEM((_NCH, _CH), jnp.int32),
            pltpu.VMEM((_NCH, _CH), jnp.int32),
            pltpu.VMEM((_NCH, _CH), jnp.int32),
            pltpu.VMEM((_CH, 2 * _D), jnp.float32),
            pltpu.VMEM((_CH, 2 * _D), jnp.float32),
            pltpu.VMEM((_CH, 128), jnp.float32),
            pltpu.VMEM((_BPW,), jnp.float32),
            pltpu.SemaphoreType.DMA,
            pltpu.SemaphoreType.DMA,
        ],
    )(hidx2d, ridx2d, tidx2d, ent_lin, cs)


def kernel(head_idx, relation_idx, tail_idx, entity_embeddings, relation_embeddings):
    cs = _trig_tables(relation_embeddings)
    tail_flat = entity_embeddings[_NT * _CH:].reshape(_TAIL * 2 * _D)
    ent_lin = _detile(entity_embeddings.T, tail_flat).reshape(_E, 2 * _D)
    h2 = head_idx.reshape(_NW * _NCH, _CH)
    r2 = relation_idx.reshape(_NW * _NCH, _CH)
    t2 = tail_idx.reshape(_NW * _NCH, _CH)
    return _score(h2, r2, t2, ent_lin, cs)


# speedup vs baseline: 2.1465x; 2.1465x over previous
import jax
import jax.numpy as jnp
from jax import lax
from jax.experimental import pallas as pl
from jax.experimental.pallas import tpu as pltpu
from jax.experimental.pallas import tpu_sc as plsc

_NC = 2
_NS = 16
_L = 16
_NW = _NC * _NS
_B = 16384
_E = 1000000
_D = 32
_BPW = _B // _NW
_CH = 128
_NCH = _BPW // _CH
_G = _CH // _L


def _trig_body(r_ref, c_ref, s_ref):
    c_ref[...] = jnp.cos(r_ref[...])
    s_ref[...] = jnp.sin(r_ref[...])


def _trig_tables(rel):
    cos_t, sin_t = pl.pallas_call(
        _trig_body,
        out_shape=(
            jax.ShapeDtypeStruct(rel.shape, rel.dtype),
            jax.ShapeDtypeStruct(rel.shape, rel.dtype),
        ),
    )(rel)
    return jnp.concatenate([cos_t, sin_t], axis=1).reshape(500, 128)


def _score_body(hidx_hbm, ridx_hbm, tidx_hbm, ent_hbm, cs_hbm, out_hbm,
                hidx_v, tidx_v, ridx_v, rsh_v,
                hrows, trows, csrows, out_v, sem, rsem):
    wid = lax.axis_index("s") * _NC + lax.axis_index("c")
    row0 = wid * _NCH
    pltpu.sync_copy(hidx_hbm.at[pl.ds(row0, _NCH)], hidx_v)
    pltpu.sync_copy(tidx_hbm.at[pl.ds(row0, _NCH)], tidx_v)
    pltpu.sync_copy(ridx_hbm.at[pl.ds(row0, _NCH)], ridx_v)
    for k in range(_NCH):
        for g in range(_G):
            sl = pl.ds(g * _L, _L)
            rsh_v[k, sl] = lax.shift_right_logical(ridx_v[k, sl], 1)

    lane = lax.iota(jnp.int32, _L)
    for k in range(_NCH):
        copies = [
            pltpu.async_copy(ent_hbm.at[hidx_v.at[k]], hrows, sem),
            pltpu.async_copy(ent_hbm.at[tidx_v.at[k]], trows, sem),
            pltpu.async_copy(cs_hbm.at[rsh_v.at[k]], csrows, rsem),
        ]
        for cp in copies:
            cp.wait()

        for g in range(_G):
            sl = pl.ds(g * _L, _L)
            rows = g * _L + lane
            pr = (ridx_v[k, sl] & 1) * 64

            def dbody(d, acc, rows=rows, pr=pr):
                dd = jnp.zeros((_L,), jnp.int32) + d
                re = plsc.load_gather(hrows, [rows, dd])
                im = plsc.load_gather(hrows, [rows, dd + _D])
                tre = plsc.load_gather(trows, [rows, dd])
                tim = plsc.load_gather(trows, [rows, dd + _D])
                c = plsc.load_gather(csrows, [rows, pr + d])
                s = plsc.load_gather(csrows, [rows, pr + d + _D])
                return (acc + jnp.abs(re * c - im * s - tre)
                        + jnp.abs(re * s + im * c - tim))

            acc = lax.fori_loop(0, _D, dbody, jnp.zeros((_L,), jnp.float32))
            out_v[pl.ds(k * _CH + g * _L, _L)] = -acc
    pltpu.sync_copy(out_v, out_hbm.at[pl.ds(wid * _BPW, _BPW)])


def _score(hidx2d, ridx2d, tidx2d, ent_pad, cs):
    mesh = plsc.VectorSubcoreMesh(
        core_axis_name="c", subcore_axis_name="s",
        num_cores=_NC, num_subcores=_NS,
    )
    return pl.kernel(
        _score_body,
        out_type=jax.ShapeDtypeStruct((_B,), jnp.float32),
        mesh=mesh,
        compiler_params=pltpu.CompilerParams(
            needs_layout_passes=False, use_tc_tiling_on_sc=False),
        scratch_types=[
            pltpu.VMEM((_NCH, _CH), jnp.int32),
            pltpu.VMEM((_NCH, _CH), jnp.int32),
            pltpu.VMEM((_NCH, _CH), jnp.int32),
            pltpu.VMEM((_NCH, _CH), jnp.int32),
            pltpu.VMEM((_CH, 128), jnp.float32),
            pltpu.VMEM((_CH, 128), jnp.float32),
            pltpu.VMEM((_CH, 128), jnp.float32),
            pltpu.VMEM((_BPW,), jnp.float32),
            pltpu.SemaphoreType.DMA,
            pltpu.SemaphoreType.DMA,
        ],
    )(hidx2d, ridx2d, tidx2d, ent_pad, cs)


def kernel(head_idx, relation_idx, tail_idx, entity_embeddings, relation_embeddings):
    cs = _trig_tables(relation_embeddings)
    ent_pad = jnp.concatenate(
        [entity_embeddings, jnp.zeros_like(entity_embeddings)], axis=1)
    h2 = head_idx.reshape(_NW * _NCH, _CH)
    r2 = relation_idx.reshape(_NW * _NCH, _CH)
    t2 = tail_idx.reshape(_NW * _NCH, _CH)
    return _score(h2, r2, t2, ent_pad, cs)
